# baseline (speedup 1.0000x reference)
import functools

import jax
import jax.numpy as jnp
from jax import lax
from jax.experimental import pallas as pl
from jax.experimental.pallas import tpu as pltpu
from jax.experimental.pallas import tpu_sc as plsc

VOCAB = 1000008
DIM = 300
B_ROWS = 4096
B_COLS = 200
NUM_IDX = B_ROWS * B_COLS

NC = 2
NS = 16
C_PER_SC = DIM // NC
J_PER_TILE = NUM_IDX // NS
GCHUNK = 3200
N_G = J_PER_TILE // GCHUNK

CHUNK = 3840
LAST_CHUNK = 1024
N_CHUNKS = 17
PER_TILE_LOAD = CHUNK * 16 + LAST_CHUNK
MAIN_N = NS * PER_TILE_LOAD
TAIL_A = 512
TAIL_B = VOCAB - MAIN_N - TAIL_A
TAIL_B_OFF = MAIN_N + TAIL_A


def _csize(k):
    return CHUNK if k < 16 else LAST_CHUNK


def _body(table_hbm, idx_hbm, tail_hbm, out_hbm,
          img, tail_sp, idx_v, buf0, buf1, buf2, buf3, cbuf, tailrow,
          isem0, isem1, isem2, isem3, fsem,
          gsem0, gsem1, gsem2, gsem3, wsem0, wsem1, wsem2, wsem3):
    sc = lax.axis_index("c")
    tid = lax.axis_index("s")
    cbase = sc * C_PER_SC
    jbase = tid * J_PER_TILE
    lbase = tid * PER_TILE_LOAD

    pltpu.sync_copy(idx_hbm.at[pl.ds(jbase, J_PER_TILE)], idx_v)

    @pl.when(tid == 0)
    def _():
        pltpu.sync_copy(tail_hbm, tail_sp)

    bufs = (buf0, buf1, buf2, buf3)
    isems = (isem0, isem1, isem2, isem3)
    gsems = (gsem0, gsem1, gsem2, gsem3)
    wsems = (wsem0, wsem1, wsem2, wsem3)

    def ind_desc(k):
        cref = cbuf.at[pl.ds(0, 1)]
        n = _csize(k)
        return pltpu.make_async_copy(
            table_hbm.at[cref, pl.ds(lbase + k * CHUNK, n)],
            bufs[k % 4].at[:, pl.ds(0, n)],
            isems[k % 4],
        )

    def fwd_desc(k):
        n = _csize(k)
        return pltpu.make_async_copy(
            bufs[k % 4].at[0, pl.ds(0, n)],
            img.at[pl.ds(lbase + k * CHUNK, n)],
            fsem,
        )

    def indA_desc():
        cref = cbuf.at[pl.ds(0, 1)]
        return pltpu.make_async_copy(
            table_hbm.at[cref, pl.ds(MAIN_N, TAIL_A)],
            buf1.at[:, pl.ds(0, TAIL_A)],
            isems[1],
        )

    tailA_fwd = pltpu.make_async_copy(
        buf1.at[0, pl.ds(0, TAIL_A)], img.at[pl.ds(MAIN_N, TAIL_A)], fsem)
    tailB_fwd = pltpu.make_async_copy(
        tailrow, img.at[pl.ds(TAIL_B_OFF, TAIL_B)], fsem)

    def start_load(c):
        cbuf[...] = jnp.full((16,), c, jnp.int32)
        for q in range(4):
            ind_desc(q).start()
        for k in range(N_CHUNKS):
            ind_desc(k).wait()
            fwd_desc(k).start()
            if k + 4 < N_CHUNKS:
                fwd_desc(k).wait()
                ind_desc(k + 4).start()

        @pl.when(tid == 15)
        def _():
            fwd_desc(13).wait()
            ia = indA_desc()
            ia.start()
            ia.wait()
            tailA_fwd.start()

        @pl.when(tid == 0)
        def _():
            tb = pltpu.make_async_copy(
                tail_sp.at[pl.ds(c * TAIL_B, TAIL_B)], tailrow, isem1)
            tb.start()
            tb.wait()
            tailB_fwd.start()

    def wait_load():
        @pl.when(tid != 15)
        def _():
            fwd_desc(13).wait()
        fwd_desc(14).wait()
        fwd_desc(15).wait()
        fwd_desc(16).wait()

        @pl.when(tid == 15)
        def _():
            tailA_fwd.wait()

        @pl.when(tid == 0)
        def _():
            tailB_fwd.wait()

    def g_desc(b):
        return pltpu.make_async_copy(
            img.at[idx_v.at[pl.ds(b * GCHUNK, GCHUNK)]],
            bufs[b % 4].at[0, pl.ds(0, GCHUNK)],
            gsems[b % 4],
        )

    def w_desc(c, b):
        return pltpu.make_async_copy(
            bufs[b % 4].at[0, pl.ds(0, GCHUNK)],
            out_hbm.at[pl.ds(c * NUM_IDX + jbase + b * GCHUNK, GCHUNK)],
            wsems[b % 4],
        )

    def iter_body(i, carry):
        c = cbase + i
        wait_load()
        plsc.subcore_barrier()
        for _rep in range(2):
            for q in range(4):
                g_desc(q).start()
            for b in range(N_G):
                g_desc(b).wait()
                w_desc(c, b).start()
                if b + 4 < N_G:
                    w_desc(c, b).wait()
                    g_desc(b + 4).start()
            for b in range(N_G - 4, N_G):
                w_desc(c, b).wait()
        plsc.subcore_barrier()

        @pl.when(i + 1 < C_PER_SC)
        def _():
            start_load(c + 1)

        return carry

    start_load(cbase)
    lax.fori_loop(0, C_PER_SC, iter_body, 0)


def _gather_t(table_t, idx_flat, tail_1d):
    mesh = plsc.VectorSubcoreMesh(core_axis_name="c", subcore_axis_name="s")
    k = functools.partial(
        pl.kernel,
        mesh=mesh,
        out_type=jax.ShapeDtypeStruct((DIM * NUM_IDX,), jnp.float32),
        scratch_types=[
            pltpu.VMEM_SHARED((VOCAB,), jnp.float32),
            pltpu.VMEM_SHARED((DIM * TAIL_B,), jnp.float32),
            pltpu.VMEM((J_PER_TILE,), jnp.int32),
            pltpu.VMEM((1, CHUNK), jnp.float32),
            pltpu.VMEM((1, CHUNK), jnp.float32),
            pltpu.VMEM((1, CHUNK), jnp.float32),
            pltpu.VMEM((1, CHUNK), jnp.float32),
            pltpu.VMEM((16,), jnp.int32),
            pltpu.VMEM((TAIL_B,), jnp.float32),
            pltpu.SemaphoreType.DMA,
            pltpu.SemaphoreType.DMA,
            pltpu.SemaphoreType.DMA,
            pltpu.SemaphoreType.DMA,
            pltpu.SemaphoreType.DMA,
            pltpu.SemaphoreType.DMA,
            pltpu.SemaphoreType.DMA,
            pltpu.SemaphoreType.DMA,
            pltpu.SemaphoreType.DMA,
            pltpu.SemaphoreType.DMA,
            pltpu.SemaphoreType.DMA,
            pltpu.SemaphoreType.DMA,
            pltpu.SemaphoreType.DMA,
        ],
    )(_body)
    return k(table_t, idx_flat, tail_1d)


def kernel(table, idxes):
    table_t = jnp.transpose(table)
    idx_flat = jnp.transpose(idxes).reshape(NUM_IDX).astype(jnp.int32)
    tail_1d = jnp.transpose(table[TAIL_B_OFF:, :]).reshape(DIM * TAIL_B)
    out_flat = _gather_t(table_t, idx_flat, tail_1d)
    out3 = out_flat.reshape(DIM, B_COLS, B_ROWS)
    return jnp.transpose(out3, (2, 1, 0))


# speedup vs baseline: 1.5272x; 1.5272x over previous
import functools

import jax
import jax.numpy as jnp
from jax import lax
from jax.experimental import pallas as pl
from jax.experimental.pallas import tpu as pltpu
from jax.experimental.pallas import tpu_sc as plsc

VOCAB = 1000008
DIM = 300
B_ROWS = 4096
B_COLS = 200
NUM_IDX = B_ROWS * B_COLS

NC = 2
NS = 16
C_PER_SC = DIM // NC
J_PER_TILE = NUM_IDX // NS
GCHUNK = 2560
N_G = J_PER_TILE // GCHUNK
NSLOT = 6
LOOK = 3

CHUNK = 2560
LAST_CHUNK = 1024
N_CHUNKS = 25
PER_TILE_LOAD = CHUNK * 24 + LAST_CHUNK
MAIN_N = NS * PER_TILE_LOAD
TAIL_A = 512
TAIL_B = VOCAB - MAIN_N - TAIL_A
TAIL_B_OFF = MAIN_N + TAIL_A


def _csize(k):
    return CHUNK if k < N_CHUNKS - 1 else LAST_CHUNK


def _body(table_hbm, idx_hbm, tail_hbm, out_hbm,
          img, tail_sp, idx_v,
          buf0, buf1, buf2, buf3, buf4, buf5, cbuf, tailrow,
          isem0, isem1, isem2, isem3, isem4, isem5, fsem,
          gsem0, gsem1, gsem2, gsem3, gsem4, gsem5,
          wsem0, wsem1, wsem2, wsem3, wsem4, wsem5):
    sc = lax.axis_index("c")
    tid = lax.axis_index("s")
    cbase = sc * C_PER_SC
    jbase = tid * J_PER_TILE
    lbase = tid * PER_TILE_LOAD

    pltpu.sync_copy(idx_hbm.at[pl.ds(jbase, J_PER_TILE)], idx_v)

    @pl.when(tid == 0)
    def _():
        pltpu.sync_copy(tail_hbm, tail_sp)

    bufs = (buf0, buf1, buf2, buf3, buf4, buf5)
    isems = (isem0, isem1, isem2, isem3, isem4, isem5)
    gsems = (gsem0, gsem1, gsem2, gsem3, gsem4, gsem5)
    wsems = (wsem0, wsem1, wsem2, wsem3, wsem4, wsem5)

    def ind_desc(k):
        cref = cbuf.at[pl.ds(0, 1)]
        n = _csize(k)
        return pltpu.make_async_copy(
            table_hbm.at[cref, pl.ds(lbase + k * CHUNK, n)],
            bufs[k % NSLOT].at[:, pl.ds(0, n)],
            isems[k % NSLOT],
        )

    def fwd_desc(k):
        n = _csize(k)
        return pltpu.make_async_copy(
            bufs[k % NSLOT].at[0, pl.ds(0, n)],
            img.at[pl.ds(lbase + k * CHUNK, n)],
            fsem,
        )

    XSLOT = (N_CHUNKS - 1) % NSLOT

    def indA_desc():
        cref = cbuf.at[pl.ds(0, 1)]
        return pltpu.make_async_copy(
            table_hbm.at[cref, pl.ds(MAIN_N, TAIL_A)],
            bufs[XSLOT].at[:, pl.ds(0, TAIL_A)],
            isems[XSLOT],
        )

    tailA_fwd = pltpu.make_async_copy(
        bufs[XSLOT].at[0, pl.ds(0, TAIL_A)],
        img.at[pl.ds(MAIN_N, TAIL_A)], fsem)
    tailB_fwd = pltpu.make_async_copy(
        tailrow, img.at[pl.ds(TAIL_B_OFF, TAIL_B)], fsem)

    def start_load(c):
        cbuf[...] = jnp.full((16,), c, jnp.int32)
        for q in range(LOOK):
            ind_desc(q).start()
        for k in range(N_CHUNKS):
            ka = k + LOOK
            if ka < N_CHUNKS:
                if ka - NSLOT >= 0:
                    fwd_desc(ka - NSLOT).wait()
                ind_desc(ka).start()
            ind_desc(k).wait()
            fwd_desc(k).start()

        @pl.when(tid == 15)
        def _():
            fwd_desc(N_CHUNKS - 1).wait()
            ia = indA_desc()
            ia.start()
            ia.wait()
            tailA_fwd.start()

        @pl.when(tid == 0)
        def _():
            tb = pltpu.make_async_copy(
                tail_sp.at[pl.ds(c * TAIL_B, TAIL_B)], tailrow, isem1)
            tb.start()
            tb.wait()
            tailB_fwd.start()

    def wait_load():
        for k in range(N_CHUNKS - NSLOT, N_CHUNKS - 1):
            fwd_desc(k).wait()

        @pl.when(tid != 15)
        def _():
            fwd_desc(N_CHUNKS - 1).wait()

        @pl.when(tid == 15)
        def _():
            tailA_fwd.wait()

        @pl.when(tid == 0)
        def _():
            tailB_fwd.wait()

    def g_desc(b):
        return pltpu.make_async_copy(
            img.at[idx_v.at[pl.ds(b * GCHUNK, GCHUNK)]],
            bufs[b % NSLOT].at[0, pl.ds(0, GCHUNK)],
            gsems[b % NSLOT],
        )

    def w_desc(c, b):
        return pltpu.make_async_copy(
            bufs[b % NSLOT].at[0, pl.ds(0, GCHUNK)],
            out_hbm.at[pl.ds(c * NUM_IDX + jbase + b * GCHUNK, GCHUNK)],
            wsems[b % NSLOT],
        )

    def iter_body(i, carry):
        c = cbase + i
        wait_load()
        plsc.subcore_barrier()
        for q in range(LOOK):
            g_desc(q).start()
        for b in range(N_G):
            nb = b + LOOK
            if nb < N_G:
                if nb - NSLOT >= 0:
                    w_desc(c, nb - NSLOT).wait()
                g_desc(nb).start()
            g_desc(b).wait()
            w_desc(c, b).start()
        for b in range(N_G - NSLOT, N_G):
            w_desc(c, b).wait()
        plsc.subcore_barrier()

        @pl.when(i + 1 < C_PER_SC)
        def _():
            start_load(c + 1)

        return carry

    start_load(cbase)
    lax.fori_loop(0, C_PER_SC, iter_body, 0)


def _gather_t(table_t, idx_flat, tail_1d):
    mesh = plsc.VectorSubcoreMesh(core_axis_name="c", subcore_axis_name="s")
    k = functools.partial(
        pl.kernel,
        mesh=mesh,
        out_type=jax.ShapeDtypeStruct((DIM * NUM_IDX,), jnp.float32),
        scratch_types=[
            pltpu.VMEM_SHARED((VOCAB,), jnp.float32),
            pltpu.VMEM_SHARED((DIM * TAIL_B,), jnp.float32),
            pltpu.VMEM((J_PER_TILE,), jnp.int32),
            pltpu.VMEM((1, CHUNK), jnp.float32),
            pltpu.VMEM((1, CHUNK), jnp.float32),
            pltpu.VMEM((1, CHUNK), jnp.float32),
            pltpu.VMEM((1, CHUNK), jnp.float32),
            pltpu.VMEM((1, CHUNK), jnp.float32),
            pltpu.VMEM((1, CHUNK), jnp.float32),
            pltpu.VMEM((16,), jnp.int32),
            pltpu.VMEM((TAIL_B,), jnp.float32),
        ] + [pltpu.SemaphoreType.DMA] * 19,
    )(_body)
    return k(table_t, idx_flat, tail_1d)


def kernel(table, idxes):
    table_t = jnp.transpose(table)
    idx_flat = jnp.transpose(idxes).reshape(NUM_IDX).astype(jnp.int32)
    tail_1d = jnp.transpose(table[TAIL_B_OFF:, :]).reshape(DIM * TAIL_B)
    out_flat = _gather_t(table_t, idx_flat, tail_1d)
    out3 = out_flat.reshape(DIM, B_COLS, B_ROWS)
    return jnp.transpose(out3, (2, 1, 0))


# speedup vs baseline: 1.5325x; 1.0035x over previous
import functools

import jax
import jax.numpy as jnp
from jax import lax
from jax.experimental import pallas as pl
from jax.experimental.pallas import tpu as pltpu
from jax.experimental.pallas import tpu_sc as plsc

VOCAB = 1000008
DIM = 300
B_ROWS = 4096
B_COLS = 200
NUM_IDX = B_ROWS * B_COLS

NC = 2
NS = 16
C_PER_SC = DIM // NC
J_PER_TILE = NUM_IDX // NS
GCHUNK = 5120
N_G = J_PER_TILE // GCHUNK
NSLOT = 3
LOOK = 2

CHUNK = 5120
LAST_CHUNK = 1024
N_CHUNKS = 13
PER_TILE_LOAD = CHUNK * 12 + LAST_CHUNK
MAIN_N = NS * PER_TILE_LOAD
TAIL_A = 512
TAIL_B = VOCAB - MAIN_N - TAIL_A
TAIL_B_OFF = MAIN_N + TAIL_A


def _csize(k):
    return CHUNK if k < N_CHUNKS - 1 else LAST_CHUNK


def _body(table_hbm, idx_hbm, tail_hbm, out_hbm,
          img, tail_sp, idx_v,
          buf0, buf1, buf2, cbuf, tailrow,
          isem0, isem1, isem2, fsem,
          gsem0, gsem1, gsem2,
          wsem0, wsem1, wsem2):
    sc = lax.axis_index("c")
    tid = lax.axis_index("s")
    cbase = sc * C_PER_SC
    jbase = tid * J_PER_TILE
    lbase = tid * PER_TILE_LOAD

    pltpu.sync_copy(idx_hbm.at[pl.ds(jbase, J_PER_TILE)], idx_v)

    @pl.when(tid == 0)
    def _():
        pltpu.sync_copy(tail_hbm, tail_sp)

    bufs = (buf0, buf1, buf2)
    isems = (isem0, isem1, isem2)
    gsems = (gsem0, gsem1, gsem2)
    wsems = (wsem0, wsem1, wsem2)

    def ind_desc(k):
        cref = cbuf.at[pl.ds(0, 1)]
        n = _csize(k)
        return pltpu.make_async_copy(
            table_hbm.at[cref, pl.ds(lbase + k * CHUNK, n)],
            bufs[k % NSLOT].at[:, pl.ds(0, n)],
            isems[k % NSLOT],
        )

    def fwd_desc(k):
        n = _csize(k)
        return pltpu.make_async_copy(
            bufs[k % NSLOT].at[0, pl.ds(0, n)],
            img.at[pl.ds(lbase + k * CHUNK, n)],
            fsem,
        )

    XSLOT = (N_CHUNKS - 1) % NSLOT

    def indA_desc():
        cref = cbuf.at[pl.ds(0, 1)]
        return pltpu.make_async_copy(
            table_hbm.at[cref, pl.ds(MAIN_N, TAIL_A)],
            bufs[XSLOT].at[:, pl.ds(0, TAIL_A)],
            isems[XSLOT],
        )

    tailA_fwd = pltpu.make_async_copy(
        bufs[XSLOT].at[0, pl.ds(0, TAIL_A)],
        img.at[pl.ds(MAIN_N, TAIL_A)], fsem)
    tailB_fwd = pltpu.make_async_copy(
        tailrow, img.at[pl.ds(TAIL_B_OFF, TAIL_B)], fsem)

    def start_load(c):
        cbuf[...] = jnp.full((16,), c, jnp.int32)
        for q in range(LOOK):
            ind_desc(q).start()
        for k in range(N_CHUNKS):
            ka = k + LOOK
            if ka < N_CHUNKS:
                if ka - NSLOT >= 0:
                    fwd_desc(ka - NSLOT).wait()
                ind_desc(ka).start()
            ind_desc(k).wait()
            fwd_desc(k).start()

        @pl.when(tid == 15)
        def _():
            fwd_desc(N_CHUNKS - 1).wait()
            ia = indA_desc()
            ia.start()
            ia.wait()
            tailA_fwd.start()

        @pl.when(tid == 0)
        def _():
            tb = pltpu.make_async_copy(
                tail_sp.at[pl.ds(c * TAIL_B, TAIL_B)], tailrow, isem1)
            tb.start()
            tb.wait()
            tailB_fwd.start()

    def wait_load():
        for k in range(N_CHUNKS - NSLOT, N_CHUNKS - 1):
            fwd_desc(k).wait()

        @pl.when(tid != 15)
        def _():
            fwd_desc(N_CHUNKS - 1).wait()

        @pl.when(tid == 15)
        def _():
            tailA_fwd.wait()

        @pl.when(tid == 0)
        def _():
            tailB_fwd.wait()

    def g_desc(b):
        return pltpu.make_async_copy(
            img.at[idx_v.at[pl.ds(b * GCHUNK, GCHUNK)]],
            bufs[b % NSLOT].at[0, pl.ds(0, GCHUNK)],
            gsems[b % NSLOT],
        )

    def w_desc(c, b):
        return pltpu.make_async_copy(
            bufs[b % NSLOT].at[0, pl.ds(0, GCHUNK)],
            out_hbm.at[pl.ds(c * NUM_IDX + jbase + b * GCHUNK, GCHUNK)],
            wsems[b % NSLOT],
        )

    def iter_body(i, carry):
        c = cbase + i
        wait_load()
        plsc.subcore_barrier()
        for q in range(LOOK):
            g_desc(q).start()
        for b in range(N_G):
            nb = b + LOOK
            if nb < N_G:
                if nb - NSLOT >= 0:
                    w_desc(c, nb - NSLOT).wait()
                g_desc(nb).start()
            g_desc(b).wait()
            w_desc(c, b).start()
        for b in range(N_G - NSLOT, N_G):
            w_desc(c, b).wait()
        plsc.subcore_barrier()

        @pl.when(i + 1 < C_PER_SC)
        def _():
            start_load(c + 1)

        return carry

    start_load(cbase)
    lax.fori_loop(0, C_PER_SC, iter_body, 0)


def _gather_t(table_t, idx_flat, tail_1d):
    mesh = plsc.VectorSubcoreMesh(core_axis_name="c", subcore_axis_name="s")
    k = functools.partial(
        pl.kernel,
        mesh=mesh,
        out_type=jax.ShapeDtypeStruct((DIM * NUM_IDX,), jnp.float32),
        scratch_types=[
            pltpu.VMEM_SHARED((VOCAB,), jnp.float32),
            pltpu.VMEM_SHARED((DIM * TAIL_B,), jnp.float32),
            pltpu.VMEM((J_PER_TILE,), jnp.int32),
            pltpu.VMEM((1, CHUNK), jnp.float32),
            pltpu.VMEM((1, CHUNK), jnp.float32),
            pltpu.VMEM((1, CHUNK), jnp.float32),
            pltpu.VMEM((16,), jnp.int32),
            pltpu.VMEM((TAIL_B,), jnp.float32),
        ] + [pltpu.SemaphoreType.DMA] * 10,
    )(_body)
    return k(table_t, idx_flat, tail_1d)


def kernel(table, idxes):
    table_t = jnp.transpose(table)
    idx_flat = jnp.transpose(idxes).reshape(NUM_IDX).astype(jnp.int32)
    tail_1d = jnp.transpose(table[TAIL_B_OFF:, :]).reshape(DIM * TAIL_B)
    out_flat = _gather_t(table_t, idx_flat, tail_1d)
    out3 = out_flat.reshape(DIM, B_COLS, B_ROWS)
    return jnp.transpose(out3, (2, 1, 0))


# speedup vs baseline: 1.5440x; 1.0075x over previous
import functools

import jax
import jax.numpy as jnp
from jax import lax
from jax.experimental import pallas as pl
from jax.experimental.pallas import tpu as pltpu
from jax.experimental.pallas import tpu_sc as plsc

VOCAB = 1000008
DIM = 300
B_ROWS = 4096
B_COLS = 200
NUM_IDX = B_ROWS * B_COLS

NC = 2
NS = 16
C_PER_SC = DIM // NC
J_PER_TILE = NUM_IDX // NS
GCHUNK = 5120
N_G = J_PER_TILE // GCHUNK
NSLOT = 3
LOOK = 2

CHUNK = 5120
LAST_CHUNK = 1024
N_CHUNKS = 13
PER_TILE_LOAD = CHUNK * 12 + LAST_CHUNK
MAIN_N = NS * PER_TILE_LOAD
TAIL_A = 512
TAIL_B = VOCAB - MAIN_N - TAIL_A
TAIL_B_OFF = MAIN_N + TAIL_A


def _csize(k):
    return CHUNK if k < N_CHUNKS - 1 else LAST_CHUNK


def _body(table_hbm, idx_hbm, tail_hbm, out_hbm,
          img, tail_sp, idx_v,
          buf0, buf1, buf2, cbuf, tailrow,
          isem0, isem1, isem2, fsem,
          gsem0, gsem1, gsem2,
          wsem0, wsem1, wsem2):
    sc = lax.axis_index("c")
    tid = lax.axis_index("s")
    cbase = sc * C_PER_SC
    jbase = tid * J_PER_TILE
    lbase = tid * PER_TILE_LOAD

    pltpu.sync_copy(idx_hbm.at[pl.ds(jbase, J_PER_TILE)], idx_v)

    @pl.when(tid == 0)
    def _():
        pltpu.sync_copy(tail_hbm, tail_sp)

    bufs = (buf0, buf1, buf2)
    isems = (isem0, isem1, isem2)
    gsems = (gsem0, gsem1, gsem2)
    wsems = (wsem0, wsem1, wsem2)

    def ind_desc(k):
        cref = cbuf.at[pl.ds(0, 1)]
        n = _csize(k)
        return pltpu.make_async_copy(
            table_hbm.at[cref, pl.ds(lbase + k * CHUNK, n)],
            bufs[k % NSLOT].at[:, pl.ds(0, n)],
            isems[k % NSLOT],
        )

    def fwd_desc(k):
        n = _csize(k)
        return pltpu.make_async_copy(
            bufs[k % NSLOT].at[0, pl.ds(0, n)],
            img.at[pl.ds(lbase + k * CHUNK, n)],
            fsem,
        )

    XSLOT = (N_CHUNKS - 1) % NSLOT

    def indA_desc():
        cref = cbuf.at[pl.ds(0, 1)]
        return pltpu.make_async_copy(
            table_hbm.at[cref, pl.ds(MAIN_N, TAIL_A)],
            bufs[XSLOT].at[:, pl.ds(0, TAIL_A)],
            isems[XSLOT],
        )

    tailA_fwd = pltpu.make_async_copy(
        bufs[XSLOT].at[0, pl.ds(0, TAIL_A)],
        img.at[pl.ds(MAIN_N, TAIL_A)], fsem)
    tailB_fwd = pltpu.make_async_copy(
        tailrow, img.at[pl.ds(TAIL_B_OFF, TAIL_B)], fsem)

    def load_prologue(c):
        cbuf[...] = jnp.full((16,), c, jnp.int32)
        for q in range(LOOK):
            ind_desc(q).start()

    def start_load(c):
        for k in range(N_CHUNKS):
            ka = k + LOOK
            if ka < N_CHUNKS:
                if ka - NSLOT >= 0:
                    fwd_desc(ka - NSLOT).wait()
                ind_desc(ka).start()
            ind_desc(k).wait()
            fwd_desc(k).start()

        @pl.when(tid == 15)
        def _():
            fwd_desc(N_CHUNKS - 1).wait()
            ia = indA_desc()
            ia.start()
            ia.wait()
            tailA_fwd.start()

        @pl.when(tid == 0)
        def _():
            tb = pltpu.make_async_copy(
                tail_sp.at[pl.ds(c * TAIL_B, TAIL_B)], tailrow, isem1)
            tb.start()
            tb.wait()
            tailB_fwd.start()

    def wait_load():
        for k in range(N_CHUNKS - NSLOT, N_CHUNKS - 1):
            fwd_desc(k).wait()

        @pl.when(tid != 15)
        def _():
            fwd_desc(N_CHUNKS - 1).wait()

        @pl.when(tid == 15)
        def _():
            tailA_fwd.wait()

        @pl.when(tid == 0)
        def _():
            tailB_fwd.wait()

    def g_desc(b):
        return pltpu.make_async_copy(
            img.at[idx_v.at[pl.ds(b * GCHUNK, GCHUNK)]],
            bufs[b % NSLOT].at[0, pl.ds(0, GCHUNK)],
            gsems[b % NSLOT],
        )

    def w_desc(c, b):
        return pltpu.make_async_copy(
            bufs[b % NSLOT].at[0, pl.ds(0, GCHUNK)],
            out_hbm.at[pl.ds(c * NUM_IDX + jbase + b * GCHUNK, GCHUNK)],
            wsems[b % NSLOT],
        )

    def iter_body(i, carry):
        c = cbase + i
        wait_load()
        plsc.subcore_barrier()
        for q in range(LOOK):
            g_desc(q).start()
        for b in range(N_G):
            nb = b + LOOK
            if nb < N_G:
                if nb - NSLOT >= 0:
                    w_desc(c, nb - NSLOT).wait()
                g_desc(nb).start()
            g_desc(b).wait()
            w_desc(c, b).start()
        for b in range(N_G - NSLOT, N_G):
            w_desc(c, b).wait()
        @pl.when(i + 1 < C_PER_SC)
        def _():
            load_prologue(c + 1)
        plsc.subcore_barrier()

        @pl.when(i + 1 < C_PER_SC)
        def _():
            start_load(c + 1)

        return carry

    load_prologue(cbase)
    start_load(cbase)
    lax.fori_loop(0, C_PER_SC, iter_body, 0)


def _gather_t(table_t, idx_flat, tail_1d):
    mesh = plsc.VectorSubcoreMesh(core_axis_name="c", subcore_axis_name="s")
    k = functools.partial(
        pl.kernel,
        mesh=mesh,
        out_type=jax.ShapeDtypeStruct((DIM * NUM_IDX,), jnp.float32),
        scratch_types=[
            pltpu.VMEM_SHARED((VOCAB,), jnp.float32),
            pltpu.VMEM_SHARED((DIM * TAIL_B,), jnp.float32),
            pltpu.VMEM((J_PER_TILE,), jnp.int32),
            pltpu.VMEM((1, CHUNK), jnp.float32),
            pltpu.VMEM((1, CHUNK), jnp.float32),
            pltpu.VMEM((1, CHUNK), jnp.float32),
            pltpu.VMEM((16,), jnp.int32),
            pltpu.VMEM((TAIL_B,), jnp.float32),
        ] + [pltpu.SemaphoreType.DMA] * 10,
    )(_body)
    return k(table_t, idx_flat, tail_1d)


def kernel(table, idxes):
    table_t = jnp.transpose(table)
    idx_flat = jnp.transpose(idxes).reshape(NUM_IDX).astype(jnp.int32)
    tail_1d = jnp.transpose(table[TAIL_B_OFF:, :]).reshape(DIM * TAIL_B)
    out_flat = _gather_t(table_t, idx_flat, tail_1d)
    out3 = out_flat.reshape(DIM, B_COLS, B_ROWS)
    return jnp.transpose(out3, (2, 1, 0))


# speedup vs baseline: 1.5518x; 1.0050x over previous
import functools

import jax
import jax.numpy as jnp
from jax import lax
from jax.experimental import pallas as pl
from jax.experimental.pallas import tpu as pltpu
from jax.experimental.pallas import tpu_sc as plsc

VOCAB = 1000008
DIM = 300
B_ROWS = 4096
B_COLS = 200
NUM_IDX = B_ROWS * B_COLS

NC = 2
NS = 16
C_PER_SC = DIM // NC
J_PER_TILE = NUM_IDX // NS
GCHUNK = 3200
N_G = J_PER_TILE // GCHUNK

CHUNK = 3840
LAST_CHUNK = 1024
N_CHUNKS = 17
PER_TILE_LOAD = CHUNK * 16 + LAST_CHUNK
MAIN_N = NS * PER_TILE_LOAD
TAIL_A = 512
TAIL_B = VOCAB - MAIN_N - TAIL_A
TAIL_B_OFF = MAIN_N + TAIL_A


def _csize(k):
    return CHUNK if k < 16 else LAST_CHUNK


def _body(table_hbm, idx_hbm, tail_hbm, out_hbm,
          img, tail_sp, idx_v, buf0, buf1, buf2, buf3, cbuf, tailrow,
          isem0, isem1, isem2, isem3, fsem,
          gsem0, gsem1, gsem2, gsem3, wsem0, wsem1, wsem2, wsem3):
    sc = lax.axis_index("c")
    tid = lax.axis_index("s")
    cbase = sc * C_PER_SC
    jbase = tid * J_PER_TILE
    lbase = tid * PER_TILE_LOAD

    pltpu.sync_copy(idx_hbm.at[pl.ds(jbase, J_PER_TILE)], idx_v)

    @pl.when(tid == 0)
    def _():
        pltpu.sync_copy(tail_hbm, tail_sp)

    bufs = (buf0, buf1, buf2, buf3)
    isems = (isem0, isem1, isem2, isem3)
    gsems = (gsem0, gsem1, gsem2, gsem3)
    wsems = (wsem0, wsem1, wsem2, wsem3)

    def ind_desc(k):
        cref = cbuf.at[pl.ds(0, 1)]
        n = _csize(k)
        return pltpu.make_async_copy(
            table_hbm.at[cref, pl.ds(lbase + k * CHUNK, n)],
            bufs[k % 4].at[:, pl.ds(0, n)],
            isems[k % 4],
        )

    def fwd_desc(k):
        n = _csize(k)
        return pltpu.make_async_copy(
            bufs[k % 4].at[0, pl.ds(0, n)],
            img.at[pl.ds(lbase + k * CHUNK, n)],
            fsem,
        )

    def indA_desc():
        cref = cbuf.at[pl.ds(0, 1)]
        return pltpu.make_async_copy(
            table_hbm.at[cref, pl.ds(MAIN_N, TAIL_A)],
            buf1.at[:, pl.ds(0, TAIL_A)],
            isems[1],
        )

    tailA_fwd = pltpu.make_async_copy(
        buf1.at[0, pl.ds(0, TAIL_A)], img.at[pl.ds(MAIN_N, TAIL_A)], fsem)
    tailB_fwd = pltpu.make_async_copy(
        tailrow, img.at[pl.ds(TAIL_B_OFF, TAIL_B)], fsem)

    def start_load(c):
        cbuf[...] = jnp.full((16,), c, jnp.int32)
        for q in range(4):
            ind_desc(q).start()
        for k in range(N_CHUNKS):
            ind_desc(k).wait()
            fwd_desc(k).start()
            if k + 4 < N_CHUNKS:
                fwd_desc(k).wait()
                ind_desc(k + 4).start()

        @pl.when(tid == 15)
        def _():
            fwd_desc(13).wait()
            ia = indA_desc()
            ia.start()
            ia.wait()
            tailA_fwd.start()

        @pl.when(tid == 0)
        def _():
            tb = pltpu.make_async_copy(
                tail_sp.at[pl.ds(c * TAIL_B, TAIL_B)], tailrow, isem1)
            tb.start()
            tb.wait()
            tailB_fwd.start()

    def wait_load():
        @pl.when(tid != 15)
        def _():
            fwd_desc(13).wait()
        fwd_desc(14).wait()
        fwd_desc(15).wait()
        fwd_desc(16).wait()

        @pl.when(tid == 15)
        def _():
            tailA_fwd.wait()

        @pl.when(tid == 0)
        def _():
            tailB_fwd.wait()

    def g_desc(b):
        return pltpu.make_async_copy(
            img.at[idx_v.at[pl.ds(b * GCHUNK, GCHUNK)]],
            bufs[b % 4].at[0, pl.ds(0, GCHUNK)],
            gsems[b % 4],
        )

    def w_desc(c, b):
        return pltpu.make_async_copy(
            bufs[b % 4].at[0, pl.ds(0, GCHUNK)],
            out_hbm.at[pl.ds(c * NUM_IDX + jbase + b * GCHUNK, GCHUNK)],
            wsems[b % 4],
        )

    def iter_body(i, carry):
        c = cbase + i
        wait_load()
        plsc.subcore_barrier()
        for q in range(4):
            g_desc(q).start()
        for b in range(N_G):
            g_desc(b).wait()
            w_desc(c, b).start()
            if b + 4 < N_G:
                w_desc(c, b).wait()
                g_desc(b + 4).start()
        for b in range(N_G - 4, N_G):
            w_desc(c, b).wait()
        plsc.subcore_barrier()

        @pl.when(i + 1 < C_PER_SC)
        def _():
            start_load(c + 1)

        return carry

    start_load(cbase)
    lax.fori_loop(0, C_PER_SC, iter_body, 0)


def _gather_t(table_t, idx_flat, tail_1d):
    mesh = plsc.VectorSubcoreMesh(core_axis_name="c", subcore_axis_name="s")
    k = functools.partial(
        pl.kernel,
        mesh=mesh,
        out_type=jax.ShapeDtypeStruct((DIM * NUM_IDX,), jnp.float32),
        scratch_types=[
            pltpu.VMEM_SHARED((VOCAB,), jnp.float32),
            pltpu.VMEM_SHARED((DIM * TAIL_B,), jnp.float32),
            pltpu.VMEM((J_PER_TILE,), jnp.int32),
            pltpu.VMEM((1, CHUNK), jnp.float32),
            pltpu.VMEM((1, CHUNK), jnp.float32),
            pltpu.VMEM((1, CHUNK), jnp.float32),
            pltpu.VMEM((1, CHUNK), jnp.float32),
            pltpu.VMEM((16,), jnp.int32),
            pltpu.VMEM((TAIL_B,), jnp.float32),
            pltpu.SemaphoreType.DMA,
            pltpu.SemaphoreType.DMA,
            pltpu.SemaphoreType.DMA,
            pltpu.SemaphoreType.DMA,
            pltpu.SemaphoreType.DMA,
            pltpu.SemaphoreType.DMA,
            pltpu.SemaphoreType.DMA,
            pltpu.SemaphoreType.DMA,
            pltpu.SemaphoreType.DMA,
            pltpu.SemaphoreType.DMA,
            pltpu.SemaphoreType.DMA,
            pltpu.SemaphoreType.DMA,
            pltpu.SemaphoreType.DMA,
        ],
    )(_body)
    return k(table_t, idx_flat, tail_1d)


def kernel(table, idxes):
    table_t = jnp.transpose(table)
    idx_flat = jnp.transpose(idxes).reshape(NUM_IDX).astype(jnp.int32)
    tail_1d = jnp.transpose(table[TAIL_B_OFF:, :]).reshape(DIM * TAIL_B)
    out_flat = _gather_t(table_t, idx_flat, tail_1d)
    out3 = out_flat.reshape(DIM, B_COLS, B_ROWS)
    return jnp.transpose(out3, (2, 1, 0))
